# baseline (speedup 1.0000x reference)
import functools

import jax
import jax.numpy as jnp
from jax import lax
from jax.experimental import pallas as pl
from jax.experimental.pallas import tpu as pltpu
from jax.experimental.pallas import tpu_sc as plsc

N = 10000
D = 128
E = 320000
NTILES = 16
NP = 10240
ROWS_PT = NP // NTILES
CHUNK = 128
GRP = 40
NROWS_E = E // CHUNK
RPT_E = 160

_sc_mesh = plsc.VectorSubcoreMesh(core_axis_name="c", subcore_axis_name="s")


def _rsqrt16(d):
    i = lax.bitcast_convert_type(d, jnp.int32)
    y = lax.bitcast_convert_type(jnp.int32(0x5F3759DF) - (i >> 1), jnp.float32)
    for _ in range(3):
        y = y * (1.5 - 0.5 * d * y * y)
    return y


@functools.partial(
    pl.kernel,
    out_type=[
        jax.ShapeDtypeStruct((2, NP, D), jnp.float32),
        jax.ShapeDtypeStruct((2, 1, NP), jnp.float32),
        jax.ShapeDtypeStruct((2, NP, D), jnp.float32),
    ],
    mesh=_sc_mesh,
    scratch_types=[
        pltpu.VMEM_SHARED((NP, D), jnp.float32),
        pltpu.VMEM_SHARED((NP,), jnp.float32),
        pltpu.VMEM_SHARED((NP,), jnp.float32),
        pltpu.VMEM((GRP, CHUNK), jnp.int32),
        pltpu.VMEM((GRP, CHUNK), jnp.int32),
        pltpu.VMEM((CHUNK, D), jnp.float32),
        pltpu.VMEM((CHUNK, D), jnp.float32),
        pltpu.VMEM((CHUNK,), jnp.float32),
        pltpu.VMEM((ROWS_PT,), jnp.float32),
        pltpu.VMEM((CHUNK,), jnp.float32),
        pltpu.VMEM((CHUNK,), jnp.float32),
        pltpu.SemaphoreType.DMA,
        pltpu.SemaphoreType.DMA,
        pltpu.SemaphoreType.DMA,
        pltpu.SemaphoreType.DMA,
    ],
)
def _sc_fused(item_hbm, user_hbm, ei2u_hbm, eu2i_hbm,
              agg_hbm, degd_hbm, h_hbm,
              agg_sh, hist_src, hist_dst, sidx_v, didx_v, stage0, stage1,
              ones_v, zbuf_v, degbuf, scalebuf, semg0, semg1, sems0, sems1):
    c = lax.axis_index("c")
    s = lax.axis_index("s")
    row0 = s * ROWS_PT

    for j in range(CHUNK // 16):
        ones_v[pl.ds(j * 16, 16)] = jnp.ones((16,), jnp.float32)

    def fill_zeros(i, carry):
        zbuf_v[pl.ds(i * 16, 16)] = jnp.zeros((16,), jnp.float32)
        return carry

    lax.fori_loop(0, ROWS_PT // 16, fill_zeros, 0)
    pltpu.sync_copy(zbuf_v, hist_src.at[pl.ds(row0, ROWS_PT)])
    pltpu.sync_copy(zbuf_v, hist_dst.at[pl.ds(row0, ROWS_PT)])

    def zero_row(i, carry):
        for j in range(D // 16):
            stage0[i, pl.ds(j * 16, 16)] = jnp.zeros((16,), jnp.float32)
        return carry

    lax.fori_loop(0, CHUNK, zero_row, 0)
    for k in range(ROWS_PT // CHUNK):
        pltpu.sync_copy(stage0, agg_sh.at[pl.ds(row0 + k * CHUNK, CHUNK)])
    plsc.subcore_barrier()

    def groups_lo():
        b = s * RPT_E
        return [(b, GRP), (b + GRP, GRP), (b + 2 * GRP, GRP), (b + 3 * GRP, GRP)]

    GROUPS_HI = [(2400, GRP), (2440, GRP), (2480, 20)]

    def deg_groups(idx_ref, groups):
        for base, size in groups:
            pltpu.sync_copy(idx_ref.at[0, pl.ds(base, size)],
                            sidx_v.at[pl.ds(0, size)])
            pltpu.sync_copy(idx_ref.at[1, pl.ds(base, size)],
                            didx_v.at[pl.ds(0, size)])

            def scat_deg(ch, carry):
                d = pltpu.async_copy(ones_v, hist_src.at[sidx_v.at[ch]],
                                     semg0, add=True)
                pltpu.sync_copy(ones_v, hist_dst.at[didx_v.at[ch]], add=True)
                d.wait()
                return carry

            lax.fori_loop(0, size, scat_deg, 0)

    def deg_phase(idx_ref):
        @pl.when(s < NTILES - 1)
        def _lo():
            deg_groups(idx_ref, groups_lo())

        @pl.when(s == NTILES - 1)
        def _hi():
            deg_groups(idx_ref, GROUPS_HI)

    @pl.when(c == 0)
    def _deg0():
        deg_phase(ei2u_hbm)

    @pl.when(c == 1)
    def _deg1():
        deg_phase(eu2i_hbm)

    plsc.subcore_barrier()

    def scale_chunk(x_ref, r0, rows):
        pltpu.sync_copy(x_ref.at[pl.ds(r0, rows)], stage0.at[pl.ds(0, rows)])
        pltpu.sync_copy(hist_src.at[pl.ds(r0, rows)], degbuf.at[pl.ds(0, rows)])

        def mkscale(i, carry):
            d = jnp.maximum(degbuf[pl.ds(i * 16, 16)], 1.0)
            scalebuf[pl.ds(i * 16, 16)] = _rsqrt16(d)
            return carry

        lax.fori_loop(0, rows // 16, mkscale, 0)

        def scalegrp(g16, carry):
            sv16 = scalebuf[pl.ds(g16 * 16, 16)]
            for lane in range(16):
                r = g16 * 16 + lane
                sv = sv16[lane]
                for j in range(D // 16):
                    stage0[r, pl.ds(j * 16, 16)] = stage0[r, pl.ds(j * 16, 16)] * sv
            return carry

        lax.fori_loop(0, rows // 16, scalegrp, 0)
        pltpu.sync_copy(stage0.at[pl.ds(0, rows)], h_hbm.at[c, pl.ds(r0, rows)])

    def scale_phase(x_ref):
        @pl.when(s < NTILES - 1)
        def _lo():
            for k in range(ROWS_PT // CHUNK):
                scale_chunk(x_ref, row0 + k * CHUNK, CHUNK)

        @pl.when(s == NTILES - 1)
        def _hi():
            for k in range(3):
                scale_chunk(x_ref, 9600 + k * CHUNK, CHUNK)
            scale_chunk(x_ref, 9984, 16)

    @pl.when(c == 0)
    def _scale0():
        scale_phase(item_hbm)

    @pl.when(c == 1)
    def _scale1():
        scale_phase(user_hbm)

    plsc.subcore_barrier()

    def agg_groups(idx_ref, groups):
        for base, size in groups:
            pltpu.sync_copy(idx_ref.at[0, pl.ds(base, size)],
                            sidx_v.at[pl.ds(0, size)])
            pltpu.sync_copy(idx_ref.at[1, pl.ds(base, size)],
                            didx_v.at[pl.ds(0, size)])
            pltpu.async_copy(h_hbm.at[c].at[sidx_v.at[0]], stage0, semg0)

            def pair(p, carry):
                ch0 = 2 * p
                ch1 = 2 * p + 1
                pltpu.make_async_copy(h_hbm.at[c].at[sidx_v.at[ch0]],
                                      stage0, semg0).wait()
                pltpu.async_copy(h_hbm.at[c].at[sidx_v.at[ch1]], stage1, semg1)
                pltpu.async_copy(stage0, agg_sh.at[didx_v.at[ch0]], sems0,
                                 add=True)
                pltpu.make_async_copy(h_hbm.at[c].at[sidx_v.at[ch1]],
                                      stage1, semg1).wait()
                pltpu.async_copy(stage1, agg_sh.at[didx_v.at[ch1]], sems1,
                                 add=True)
                pltpu.make_async_copy(stage0, agg_sh.at[didx_v.at[ch0]],
                                      sems0).wait()

                @pl.when(p < size // 2 - 1)
                def _prefetch_even():
                    pltpu.async_copy(h_hbm.at[c].at[sidx_v.at[ch0 + 2]],
                                     stage0, semg0)

                pltpu.make_async_copy(stage1, agg_sh.at[didx_v.at[ch1]],
                                      sems1).wait()
                return carry

            lax.fori_loop(0, size // 2, pair, 0)

    def agg_phase(idx_ref):
        @pl.when(s < NTILES - 1)
        def _lo():
            agg_groups(idx_ref, groups_lo())

        @pl.when(s == NTILES - 1)
        def _hi():
            agg_groups(idx_ref, GROUPS_HI)

    @pl.when(c == 0)
    def _agg0():
        agg_phase(ei2u_hbm)

    @pl.when(c == 1)
    def _agg1():
        agg_phase(eu2i_hbm)

    plsc.subcore_barrier()
    pltpu.sync_copy(agg_sh.at[pl.ds(row0, ROWS_PT)],
                    agg_hbm.at[c, pl.ds(row0, ROWS_PT)])
    pltpu.sync_copy(hist_dst.at[pl.ds(row0, ROWS_PT)],
                    degd_hbm.at[c, 0, pl.ds(row0, ROWS_PT)])


_FBLK = 1000


def _tc_final_body(agg_ref, deg_ref, w_ref, b_ref, out0_ref, out1_ref):
    a0 = agg_ref[0] * lax.rsqrt(jnp.maximum(deg_ref[0], 1.0))
    out0_ref[...] = jnp.dot(a0, w_ref[0], preferred_element_type=jnp.float32) + b_ref[0]
    a1 = agg_ref[1] * lax.rsqrt(jnp.maximum(deg_ref[1], 1.0))
    out1_ref[...] = jnp.dot(a1, w_ref[1], preferred_element_type=jnp.float32) + b_ref[1]


def _tc_final(agg, degs_dst, W, b):
    return pl.pallas_call(
        _tc_final_body,
        grid=(N // _FBLK,),
        in_specs=[
            pl.BlockSpec((2, _FBLK, D), lambda i: (0, i, 0)),
            pl.BlockSpec((2, _FBLK, 1), lambda i: (0, i, 0)),
            pl.BlockSpec((2, D, D), lambda i: (0, 0, 0)),
            pl.BlockSpec((2, 1, D), lambda i: (0, 0, 0)),
        ],
        out_specs=[
            pl.BlockSpec((_FBLK, D), lambda i: (i, 0)),
            pl.BlockSpec((_FBLK, D), lambda i: (i, 0)),
        ],
        out_shape=[
            jax.ShapeDtypeStruct((N, D), jnp.float32),
            jax.ShapeDtypeStruct((N, D), jnp.float32),
        ],
    )(agg, degs_dst, W, b)


def kernel(item_feat, user_feat, edge_index_i2u, edge_index_u2i,
           W_i2u, b_i2u, W_u2i, b_u2i):
    ei2u = edge_index_i2u.astype(jnp.int32).reshape(2, NROWS_E, CHUNK)
    eu2i = edge_index_u2i.astype(jnp.int32).reshape(2, NROWS_E, CHUNK)

    agg, deg_dst, _h = _sc_fused(item_feat, user_feat, ei2u, eu2i)
    degs_dst = deg_dst.reshape(2, NP, 1)

    W = jnp.stack([W_i2u, W_u2i])
    b = jnp.stack([b_i2u, b_u2i])[:, None, :]
    user_out, item_out = _tc_final(agg, degs_dst, W, b)
    return user_out, item_out


# speedup vs baseline: 1.1209x; 1.1209x over previous
import functools

import jax
import jax.numpy as jnp
from jax import lax
from jax.experimental import pallas as pl
from jax.experimental.pallas import tpu as pltpu
from jax.experimental.pallas import tpu_sc as plsc

N = 10000
D = 128
E = 320000
NTILES = 16
NP = 10240
ROWS_PT = NP // NTILES
CHUNK = 128
GRP = 40
NROWS_E = E // CHUNK
RPT_E = 160

_sc_mesh = plsc.VectorSubcoreMesh(core_axis_name="c", subcore_axis_name="s")


def _rsqrt16(d):
    i = lax.bitcast_convert_type(d, jnp.int32)
    y = lax.bitcast_convert_type(jnp.int32(0x5F3759DF) - (i >> 1), jnp.float32)
    for _ in range(3):
        y = y * (1.5 - 0.5 * d * y * y)
    return y


@functools.partial(
    pl.kernel,
    out_type=[
        jax.ShapeDtypeStruct((2, NP, D), jnp.float32),
        jax.ShapeDtypeStruct((2, 1, NP), jnp.float32),
        jax.ShapeDtypeStruct((2, NP, D), jnp.float32),
    ],
    mesh=_sc_mesh,
    scratch_types=[
        pltpu.VMEM_SHARED((NP, D), jnp.float32),
        pltpu.VMEM_SHARED((NP,), jnp.float32),
        pltpu.VMEM_SHARED((NP,), jnp.float32),
        pltpu.VMEM((GRP, CHUNK), jnp.int32),
        pltpu.VMEM((GRP, CHUNK), jnp.int32),
        pltpu.VMEM((CHUNK, D), jnp.float32),
        pltpu.VMEM((CHUNK, D), jnp.float32),
        pltpu.VMEM((CHUNK,), jnp.float32),
        pltpu.VMEM((ROWS_PT,), jnp.float32),
        pltpu.VMEM((CHUNK,), jnp.float32),
        pltpu.VMEM((CHUNK,), jnp.float32),
        pltpu.SemaphoreType.DMA,
        pltpu.SemaphoreType.DMA,
    ],
)
def _sc_fused(item_hbm, user_hbm, ei2u_hbm, eu2i_hbm,
              agg_hbm, degd_hbm, h_hbm,
              agg_sh, hist_src, hist_dst, sidx_v, didx_v, stage0, stage1,
              ones_v, zbuf_v, degbuf, scalebuf, semg0, semg1):
    c = lax.axis_index("c")
    s = lax.axis_index("s")
    row0 = s * ROWS_PT

    for j in range(CHUNK // 16):
        ones_v[pl.ds(j * 16, 16)] = jnp.ones((16,), jnp.float32)

    def fill_zeros(i, carry):
        zbuf_v[pl.ds(i * 16, 16)] = jnp.zeros((16,), jnp.float32)
        return carry

    lax.fori_loop(0, ROWS_PT // 16, fill_zeros, 0)
    pltpu.sync_copy(zbuf_v, hist_src.at[pl.ds(row0, ROWS_PT)])
    pltpu.sync_copy(zbuf_v, hist_dst.at[pl.ds(row0, ROWS_PT)])

    def zero_row(i, carry):
        for j in range(D // 16):
            stage0[i, pl.ds(j * 16, 16)] = jnp.zeros((16,), jnp.float32)
        return carry

    lax.fori_loop(0, CHUNK, zero_row, 0)
    for k in range(ROWS_PT // CHUNK):
        pltpu.sync_copy(stage0, agg_sh.at[pl.ds(row0 + k * CHUNK, CHUNK)])
    plsc.subcore_barrier()

    def groups_lo():
        b = s * RPT_E
        return [(b, GRP), (b + GRP, GRP), (b + 2 * GRP, GRP), (b + 3 * GRP, GRP)]

    GROUPS_HI = [(2400, GRP), (2440, GRP), (2480, 20)]

    def deg_groups(idx_ref, groups):
        for base, size in groups:
            pltpu.sync_copy(idx_ref.at[0, pl.ds(base, size)],
                            sidx_v.at[pl.ds(0, size)])
            pltpu.sync_copy(idx_ref.at[1, pl.ds(base, size)],
                            didx_v.at[pl.ds(0, size)])

            def scat_deg(ch, carry):
                d = pltpu.async_copy(ones_v, hist_src.at[sidx_v.at[ch]],
                                     semg0, add=True)
                pltpu.sync_copy(ones_v, hist_dst.at[didx_v.at[ch]], add=True)
                d.wait()
                return carry

            lax.fori_loop(0, size, scat_deg, 0)

    def deg_phase(idx_ref):
        @pl.when(s < NTILES - 1)
        def _lo():
            deg_groups(idx_ref, groups_lo())

        @pl.when(s == NTILES - 1)
        def _hi():
            deg_groups(idx_ref, GROUPS_HI)

    @pl.when(c == 0)
    def _deg0():
        deg_phase(ei2u_hbm)

    @pl.when(c == 1)
    def _deg1():
        deg_phase(eu2i_hbm)

    plsc.subcore_barrier()

    def scale_chunk(x_ref, r0, rows):
        pltpu.sync_copy(x_ref.at[pl.ds(r0, rows)], stage0.at[pl.ds(0, rows)])
        pltpu.sync_copy(hist_src.at[pl.ds(r0, rows)], degbuf.at[pl.ds(0, rows)])

        def mkscale(i, carry):
            d = jnp.maximum(degbuf[pl.ds(i * 16, 16)], 1.0)
            scalebuf[pl.ds(i * 16, 16)] = _rsqrt16(d)
            return carry

        lax.fori_loop(0, rows // 16, mkscale, 0)

        def scalegrp(g16, carry):
            sv16 = scalebuf[pl.ds(g16 * 16, 16)]
            for lane in range(16):
                r = g16 * 16 + lane
                sv = sv16[lane]
                for j in range(D // 16):
                    stage0[r, pl.ds(j * 16, 16)] = stage0[r, pl.ds(j * 16, 16)] * sv
            return carry

        lax.fori_loop(0, rows // 16, scalegrp, 0)
        pltpu.sync_copy(stage0.at[pl.ds(0, rows)], h_hbm.at[c, pl.ds(r0, rows)])

    def scale_phase(x_ref):
        @pl.when(s < NTILES - 1)
        def _lo():
            for k in range(ROWS_PT // CHUNK):
                scale_chunk(x_ref, row0 + k * CHUNK, CHUNK)

        @pl.when(s == NTILES - 1)
        def _hi():
            for k in range(3):
                scale_chunk(x_ref, 9600 + k * CHUNK, CHUNK)
            scale_chunk(x_ref, 9984, 16)

    @pl.when(c == 0)
    def _scale0():
        scale_phase(item_hbm)

    @pl.when(c == 1)
    def _scale1():
        scale_phase(user_hbm)

    plsc.subcore_barrier()

    def agg_groups(idx_ref, groups):
        for base, size in groups:
            pltpu.sync_copy(idx_ref.at[0, pl.ds(base, size)],
                            sidx_v.at[pl.ds(0, size)])
            pltpu.sync_copy(idx_ref.at[1, pl.ds(base, size)],
                            didx_v.at[pl.ds(0, size)])
            pltpu.async_copy(h_hbm.at[c].at[sidx_v.at[0]], stage0, semg0)

            def pair(p, carry):
                ch0 = 2 * p
                ch1 = 2 * p + 1
                pltpu.make_async_copy(h_hbm.at[c].at[sidx_v.at[ch0]],
                                      stage0, semg0).wait()
                pltpu.async_copy(h_hbm.at[c].at[sidx_v.at[ch1]], stage1, semg1)
                pltpu.sync_copy(stage0, agg_sh.at[didx_v.at[ch0]], add=True)

                @pl.when(p < size // 2 - 1)
                def _prefetch_even():
                    pltpu.async_copy(h_hbm.at[c].at[sidx_v.at[ch0 + 2]],
                                     stage0, semg0)

                pltpu.make_async_copy(h_hbm.at[c].at[sidx_v.at[ch1]],
                                      stage1, semg1).wait()
                pltpu.sync_copy(stage1, agg_sh.at[didx_v.at[ch1]], add=True)
                return carry

            lax.fori_loop(0, size // 2, pair, 0)

    def agg_phase(idx_ref):
        @pl.when(s < NTILES - 1)
        def _lo():
            agg_groups(idx_ref, groups_lo())

        @pl.when(s == NTILES - 1)
        def _hi():
            agg_groups(idx_ref, GROUPS_HI)

    @pl.when(c == 0)
    def _agg0():
        agg_phase(ei2u_hbm)

    @pl.when(c == 1)
    def _agg1():
        agg_phase(eu2i_hbm)

    plsc.subcore_barrier()
    pltpu.sync_copy(agg_sh.at[pl.ds(row0, ROWS_PT)],
                    agg_hbm.at[c, pl.ds(row0, ROWS_PT)])
    pltpu.sync_copy(hist_dst.at[pl.ds(row0, ROWS_PT)],
                    degd_hbm.at[c, 0, pl.ds(row0, ROWS_PT)])


_FBLK = 1000


def _tc_final_body(agg_ref, deg_ref, w_ref, b_ref, out0_ref, out1_ref):
    a0 = agg_ref[0] * lax.rsqrt(jnp.maximum(deg_ref[0], 1.0))
    out0_ref[...] = jnp.dot(a0, w_ref[0], preferred_element_type=jnp.float32) + b_ref[0]
    a1 = agg_ref[1] * lax.rsqrt(jnp.maximum(deg_ref[1], 1.0))
    out1_ref[...] = jnp.dot(a1, w_ref[1], preferred_element_type=jnp.float32) + b_ref[1]


def _tc_final(agg, degs_dst, W, b):
    return pl.pallas_call(
        _tc_final_body,
        grid=(N // _FBLK,),
        in_specs=[
            pl.BlockSpec((2, _FBLK, D), lambda i: (0, i, 0)),
            pl.BlockSpec((2, _FBLK, 1), lambda i: (0, i, 0)),
            pl.BlockSpec((2, D, D), lambda i: (0, 0, 0)),
            pl.BlockSpec((2, 1, D), lambda i: (0, 0, 0)),
        ],
        out_specs=[
            pl.BlockSpec((_FBLK, D), lambda i: (i, 0)),
            pl.BlockSpec((_FBLK, D), lambda i: (i, 0)),
        ],
        out_shape=[
            jax.ShapeDtypeStruct((N, D), jnp.float32),
            jax.ShapeDtypeStruct((N, D), jnp.float32),
        ],
    )(agg, degs_dst, W, b)


def kernel(item_feat, user_feat, edge_index_i2u, edge_index_u2i,
           W_i2u, b_i2u, W_u2i, b_u2i):
    ei2u = edge_index_i2u.astype(jnp.int32).reshape(2, NROWS_E, CHUNK)
    eu2i = edge_index_u2i.astype(jnp.int32).reshape(2, NROWS_E, CHUNK)

    agg, deg_dst, _h = _sc_fused(item_feat, user_feat, ei2u, eu2i)
    degs_dst = deg_dst.reshape(2, NP, 1)

    W = jnp.stack([W_i2u, W_u2i])
    b = jnp.stack([b_i2u, b_u2i])[:, None, :]
    user_out, item_out = _tc_final(agg, degs_dst, W, b)
    return user_out, item_out
